# Initial kernel scaffold; baseline (speedup 1.0000x reference)
import jax
import jax.numpy as jnp
from jax.experimental import pallas as pl


def kernel(x, edge_index, edge_attr, W1, b1, W2, b2, Wl, bl):
    raise NotImplementedError("write your pallas kernel here")


# speedup vs baseline: 27.6378x; 27.6378x over previous
import functools

import jax
import jax.numpy as jnp
from jax import lax
from jax.experimental import pallas as pl
from jax.experimental.pallas import tpu as pltpu
from jax.experimental.pallas import tpu_sc as plsc

N = 10000
NP = 10240
E = 320000
NTILES = 32
EPT = E // NTILES
CH = 80
NCH = EPT // CH
ROWS_PT = NP // 16
F = 16

_mesh = plsc.VectorSubcoreMesh(core_axis_name="c", subcore_axis_name="s")
_sc_params = pltpu.CompilerParams(use_tc_tiling_on_sc=False)


@functools.partial(
    pl.kernel,
    out_type=jax.ShapeDtypeStruct((2 * NP,), jnp.float32),
    mesh=_mesh,
    compiler_params=_sc_params,
    scratch_types=[
        pltpu.VMEM((NCH, CH), jnp.int32),
        pltpu.VMEM((NCH, CH), jnp.float32),
        pltpu.VMEM((ROWS_PT,), jnp.float32),
        pltpu.VMEM_SHARED((NP,), jnp.float32),
    ],
)
def _deg_kernel(dst_hbm, ew_hbm, out_hbm, dst_v, ew_v, buf_v, deg_sh):
    cid = lax.axis_index("c")
    sid = lax.axis_index("s")
    wid = cid * 16 + sid
    pltpu.sync_copy(dst_hbm.at[wid], dst_v)
    pltpu.sync_copy(ew_hbm.at[wid], ew_v)

    def zb(i, _):
        buf_v[pl.ds(i * 16, 16)] = jnp.zeros((16,), jnp.float32)
        return 0

    lax.fori_loop(0, ROWS_PT // 16, zb, 0)
    pltpu.sync_copy(buf_v, deg_sh.at[pl.ds(sid * ROWS_PT, ROWS_PT)])
    plsc.subcore_barrier()

    def body(c, _):
        pltpu.sync_copy(ew_v.at[c], deg_sh.at[dst_v.at[c]], add=True)
        return 0

    lax.fori_loop(0, NCH, body, 0)
    plsc.subcore_barrier()
    pltpu.sync_copy(deg_sh.at[pl.ds(sid * ROWS_PT, ROWS_PT)], buf_v)
    pltpu.sync_copy(buf_v, out_hbm.at[pl.ds(cid * NP + sid * ROWS_PT, ROWS_PT)])


@functools.partial(
    pl.kernel,
    out_type=jax.ShapeDtypeStruct((2 * NP, F), jnp.float32),
    mesh=_mesh,
    compiler_params=_sc_params,
    scratch_types=[
        pltpu.VMEM((NCH, CH), jnp.int32),
        pltpu.VMEM((NCH, CH), jnp.int32),
        pltpu.VMEM((NCH, CH), jnp.float32),
        pltpu.VMEM((CH, F), jnp.float32),
        pltpu.VMEM((ROWS_PT, F), jnp.float32),
        pltpu.VMEM_SHARED((NP, F), jnp.float32),
        pltpu.SemaphoreType.DMA,
    ],
)
def _agg_kernel(y_hbm, src_hbm, dst_hbm, ew_hbm, out_hbm,
                src_v, dst_v, ew_v, rows_v, buf_v, acc_sh, sem):
    cid = lax.axis_index("c")
    sid = lax.axis_index("s")
    wid = cid * 16 + sid
    pltpu.sync_copy(src_hbm.at[wid], src_v)
    pltpu.sync_copy(dst_hbm.at[wid], dst_v)
    pltpu.sync_copy(ew_hbm.at[wid], ew_v)

    def zb(i, _):
        buf_v[i, :] = jnp.zeros((16,), jnp.float32)
        return 0

    lax.fori_loop(0, ROWS_PT, zb, 0)
    pltpu.sync_copy(buf_v, acc_sh.at[pl.ds(sid * ROWS_PT, ROWS_PT)])
    plsc.subcore_barrier()

    def chunk_body(c, _):
        pltpu.async_copy(y_hbm.at[src_v.at[c]], rows_v, sem).wait()

        def scale_grp(g, _):
            wv = ew_v[c, pl.ds(g * 16, 16)]
            base = g * 16
            for l in range(16):
                r = base + l
                rows_v[r, :] = rows_v[r, :] * wv[l]
            return 0

        lax.fori_loop(0, CH // 16, scale_grp, 0)
        pltpu.sync_copy(rows_v, acc_sh.at[dst_v.at[c]], add=True)
        return 0

    lax.fori_loop(0, NCH, chunk_body, 0)
    plsc.subcore_barrier()
    pltpu.sync_copy(acc_sh.at[pl.ds(sid * ROWS_PT, ROWS_PT)], buf_v)
    pltpu.sync_copy(buf_v, out_hbm.at[pl.ds(cid * NP + sid * ROWS_PT, ROWS_PT)])


def _tc1(x, W1, degp):
    def body(x_ref, w_ref, degp_ref, y1_ref, dinv_ref):
        deg = degp_ref[0:N, :] + degp_ref[NP:NP + N, :] + 1.0
        dinv = lax.rsqrt(deg)
        xw = jnp.dot(x_ref[...], w_ref[...], preferred_element_type=jnp.float32,
                     precision=lax.Precision.HIGHEST)
        y1_ref[...] = xw * dinv
        dinv_ref[...] = dinv

    return pl.pallas_call(
        body,
        out_shape=(jax.ShapeDtypeStruct((N, F), jnp.float32),
                   jax.ShapeDtypeStruct((N, 1), jnp.float32)),
    )(x, W1, degp.reshape(2 * NP, 1))


def _tc2(aggp1, y1, dinv, b1, W2):
    def body(aggp_ref, y1_ref, dinv_ref, b1_ref, w2_ref, y2p_ref):
        agg = aggp_ref[0:N, :] + aggp_ref[NP:NP + N, :] + y1_ref[...]
        h1 = jnp.maximum(agg * dinv_ref[...] + b1_ref[...], 0.0)
        y2 = jnp.dot(h1, w2_ref[...], preferred_element_type=jnp.float32,
                     precision=lax.Precision.HIGHEST)
        y2 = y2 * dinv_ref[...]
        y2p_ref[...] = jnp.concatenate(
            [y2, jnp.zeros((N, F - 4), jnp.float32)], axis=1)

    return pl.pallas_call(
        body,
        out_shape=jax.ShapeDtypeStruct((N, F), jnp.float32),
    )(aggp1, y1, dinv, b1, W2)


def _tc3(aggp2, y2p, dinv, b2, Wl, bl):
    def body(aggp_ref, y2p_ref, dinv_ref, b2_ref, wl_ref, bl_ref, out_ref):
        agg = (aggp_ref[0:N, 0:4] + aggp_ref[NP:NP + N, 0:4]
               + y2p_ref[0:N, 0:4])
        h2 = jnp.maximum(agg * dinv_ref[...] + b2_ref[...], 0.0)
        out = jnp.dot(h2, wl_ref[...], preferred_element_type=jnp.float32,
                     precision=lax.Precision.HIGHEST)
        out_ref[...] = out + bl_ref[...]

    return pl.pallas_call(
        body,
        out_shape=jax.ShapeDtypeStruct((N, 1), jnp.float32),
    )(aggp2, y2p, dinv, b2, Wl, bl)


def kernel(x, edge_index, edge_attr, W1, b1, W2, b2, Wl, bl):
    src = edge_index[0].reshape(NTILES, NCH, CH)
    dst = edge_index[1].reshape(NTILES, NCH, CH)
    ew3d = edge_attr.reshape(NTILES, NCH, CH)
    degp = _deg_kernel(dst, ew3d)
    y1, dinv = _tc1(x, W1, degp)
    aggp1 = _agg_kernel(y1, src, dst, ew3d)
    y2p = _tc2(aggp1, y1, dinv, b1.reshape(1, F), W2)
    aggp2 = _agg_kernel(y2p, src, dst, ew3d)
    return _tc3(aggp2, y2p, dinv, b2.reshape(1, 4), Wl, bl.reshape(1, 1))


# speedup vs baseline: 39.2403x; 1.4198x over previous
import functools

import jax
import jax.numpy as jnp
from jax import lax
from jax.experimental import pallas as pl
from jax.experimental.pallas import tpu as pltpu
from jax.experimental.pallas import tpu_sc as plsc

N = 10000
NP = 10240
E = 320000
NTILES = 32
EPT = E // NTILES
CH = 80
NCH = EPT // CH
ROWS_PT = NP // 16
F = 16

_mesh = plsc.VectorSubcoreMesh(core_axis_name="c", subcore_axis_name="s")
_sc_params = pltpu.CompilerParams(use_tc_tiling_on_sc=False)


@functools.partial(
    pl.kernel,
    out_type=jax.ShapeDtypeStruct((2 * NP,), jnp.float32),
    mesh=_mesh,
    compiler_params=_sc_params,
    scratch_types=[
        pltpu.VMEM((NCH, CH), jnp.int32),
        pltpu.VMEM((NCH, CH), jnp.float32),
        pltpu.VMEM((ROWS_PT,), jnp.float32),
        pltpu.VMEM_SHARED((NP,), jnp.float32),
    ],
)
def _deg_kernel(dst_hbm, ew_hbm, out_hbm, dst_v, ew_v, buf_v, deg_sh):
    cid = lax.axis_index("c")
    sid = lax.axis_index("s")
    wid = cid * 16 + sid
    pltpu.sync_copy(dst_hbm.at[wid], dst_v)
    pltpu.sync_copy(ew_hbm.at[wid], ew_v)

    def zb(i, _):
        buf_v[pl.ds(i * 16, 16)] = jnp.zeros((16,), jnp.float32)
        return 0

    lax.fori_loop(0, ROWS_PT // 16, zb, 0)
    pltpu.sync_copy(buf_v, deg_sh.at[pl.ds(sid * ROWS_PT, ROWS_PT)])
    plsc.subcore_barrier()

    def body(c, _):
        pltpu.sync_copy(ew_v.at[c], deg_sh.at[dst_v.at[c]], add=True)
        return 0

    lax.fori_loop(0, NCH, body, 0)
    plsc.subcore_barrier()
    pltpu.sync_copy(deg_sh.at[pl.ds(sid * ROWS_PT, ROWS_PT)], buf_v)
    pltpu.sync_copy(buf_v, out_hbm.at[pl.ds(cid * NP + sid * ROWS_PT, ROWS_PT)])


@functools.partial(
    pl.kernel,
    out_type=jax.ShapeDtypeStruct((2 * NP, F), jnp.float32),
    mesh=_mesh,
    compiler_params=_sc_params,
    scratch_types=[
        pltpu.VMEM((NCH, CH), jnp.int32),
        pltpu.VMEM((NCH, CH), jnp.int32),
        pltpu.VMEM((NCH, CH), jnp.float32),
        pltpu.VMEM((CH, F), jnp.float32),
        pltpu.VMEM((CH, F), jnp.float32),
        pltpu.VMEM((ROWS_PT, F), jnp.float32),
        pltpu.VMEM_SHARED((NP, F), jnp.float32),
        pltpu.SemaphoreType.DMA,
        pltpu.SemaphoreType.DMA,
    ],
)
def _agg_kernel(y_hbm, src_hbm, dst_hbm, ew_hbm, out_hbm,
                src_v, dst_v, ew_v, rows0_v, rows1_v, buf_v, acc_sh,
                sem0, sem1):
    cid = lax.axis_index("c")
    sid = lax.axis_index("s")
    wid = cid * 16 + sid
    pltpu.sync_copy(src_hbm.at[wid], src_v)
    pltpu.sync_copy(dst_hbm.at[wid], dst_v)
    pltpu.sync_copy(ew_hbm.at[wid], ew_v)

    def zb(i, _):
        buf_v[i, :] = jnp.zeros((16,), jnp.float32)
        return 0

    lax.fori_loop(0, ROWS_PT, zb, 0)
    pltpu.sync_copy(buf_v, acc_sh.at[pl.ds(sid * ROWS_PT, ROWS_PT)])
    plsc.subcore_barrier()

    def scale(c, rows_ref):
        def scale_grp(g, _):
            wv = ew_v[c, pl.ds(g * 16, 16)]
            base = g * 16
            for l in range(16):
                r = base + l
                rows_ref[r, :] = rows_ref[r, :] * wv[l]
            return 0

        lax.fori_loop(0, CH // 16, scale_grp, 0)

    pltpu.async_copy(y_hbm.at[src_v.at[0]], rows0_v, sem0)

    def pair_body(p, _):
        c0 = 2 * p
        pltpu.async_copy(y_hbm.at[src_v.at[c0 + 1]], rows1_v, sem1)
        pltpu.make_async_copy(y_hbm.at[src_v.at[c0]], rows0_v, sem0).wait()
        scale(c0, rows0_v)
        pltpu.sync_copy(rows0_v, acc_sh.at[dst_v.at[c0]], add=True)
        pltpu.async_copy(y_hbm.at[src_v.at[c0 + 2]], rows0_v, sem0)
        pltpu.make_async_copy(y_hbm.at[src_v.at[c0 + 1]], rows1_v, sem1).wait()
        scale(c0 + 1, rows1_v)
        pltpu.sync_copy(rows1_v, acc_sh.at[dst_v.at[c0 + 1]], add=True)
        return 0

    lax.fori_loop(0, (NCH - 1) // 2, pair_body, 0)
    cl = NCH - 1
    pltpu.make_async_copy(y_hbm.at[src_v.at[cl]], rows0_v, sem0).wait()
    scale(cl, rows0_v)
    pltpu.sync_copy(rows0_v, acc_sh.at[dst_v.at[cl]], add=True)
    plsc.subcore_barrier()
    pltpu.sync_copy(acc_sh.at[pl.ds(sid * ROWS_PT, ROWS_PT)], buf_v)
    pltpu.sync_copy(buf_v, out_hbm.at[pl.ds(cid * NP + sid * ROWS_PT, ROWS_PT)])


def _tc1(x, W1, degp):
    def body(x_ref, w_ref, degp_ref, y1_ref, dinv_ref):
        deg = degp_ref[0:N, :] + degp_ref[NP:NP + N, :] + 1.0
        dinv = lax.rsqrt(deg)
        xw = jnp.dot(x_ref[...], w_ref[...], preferred_element_type=jnp.float32,
                     precision=lax.Precision.HIGHEST)
        y1_ref[...] = xw * dinv
        dinv_ref[...] = dinv

    return pl.pallas_call(
        body,
        out_shape=(jax.ShapeDtypeStruct((N, F), jnp.float32),
                   jax.ShapeDtypeStruct((N, 1), jnp.float32)),
    )(x, W1, degp.reshape(2 * NP, 1))


def _tc2(aggp1, y1, dinv, b1, W2):
    def body(aggp_ref, y1_ref, dinv_ref, b1_ref, w2_ref, y2p_ref):
        agg = aggp_ref[0:N, :] + aggp_ref[NP:NP + N, :] + y1_ref[...]
        h1 = jnp.maximum(agg * dinv_ref[...] + b1_ref[...], 0.0)
        y2 = jnp.dot(h1, w2_ref[...], preferred_element_type=jnp.float32,
                     precision=lax.Precision.HIGHEST)
        y2 = y2 * dinv_ref[...]
        y2p_ref[...] = jnp.concatenate(
            [y2, jnp.zeros((N, F - 4), jnp.float32)], axis=1)

    return pl.pallas_call(
        body,
        out_shape=jax.ShapeDtypeStruct((N, F), jnp.float32),
    )(aggp1, y1, dinv, b1, W2)


def _tc3(aggp2, y2p, dinv, b2, Wl, bl):
    def body(aggp_ref, y2p_ref, dinv_ref, b2_ref, wl_ref, bl_ref, out_ref):
        agg = (aggp_ref[0:N, 0:4] + aggp_ref[NP:NP + N, 0:4]
               + y2p_ref[0:N, 0:4])
        h2 = jnp.maximum(agg * dinv_ref[...] + b2_ref[...], 0.0)
        out = jnp.dot(h2, wl_ref[...], preferred_element_type=jnp.float32,
                     precision=lax.Precision.HIGHEST)
        out_ref[...] = out + bl_ref[...]

    return pl.pallas_call(
        body,
        out_shape=jax.ShapeDtypeStruct((N, 1), jnp.float32),
    )(aggp2, y2p, dinv, b2, Wl, bl)


def kernel(x, edge_index, edge_attr, W1, b1, W2, b2, Wl, bl):
    src = edge_index[0].reshape(NTILES, NCH, CH)
    dst = edge_index[1].reshape(NTILES, NCH, CH)
    ew3d = edge_attr.reshape(NTILES, NCH, CH)
    degp = _deg_kernel(dst, ew3d)
    y1, dinv = _tc1(x, W1, degp)
    aggp1 = _agg_kernel(y1, src, dst, ew3d)
    y2p = _tc2(aggp1, y1, dinv, b1.reshape(1, F), W2)
    aggp2 = _agg_kernel(y2p, src, dst, ew3d)
    return _tc3(aggp2, y2p, dinv, b2.reshape(1, 4), Wl, bl.reshape(1, 1))


# speedup vs baseline: 39.9346x; 1.0177x over previous
import functools

import jax
import jax.numpy as jnp
from jax import lax
from jax.experimental import pallas as pl
from jax.experimental.pallas import tpu as pltpu
from jax.experimental.pallas import tpu_sc as plsc

N = 10000
NP = 10240
E = 320000
NTILES = 32
CH = 128
NCH = 80
EPT = NCH * CH
ROWS_PT = NP // 16
F = 16

_mesh = plsc.VectorSubcoreMesh(core_axis_name="c", subcore_axis_name="s")
_sc_params = pltpu.CompilerParams(use_tc_tiling_on_sc=False)


@functools.partial(
    pl.kernel,
    out_type=jax.ShapeDtypeStruct((2 * NP,), jnp.float32),
    mesh=_mesh,
    compiler_params=_sc_params,
    scratch_types=[
        pltpu.VMEM((NCH, CH), jnp.int32),
        pltpu.VMEM((NCH, CH), jnp.float32),
        pltpu.VMEM_SHARED((NP,), jnp.float32),
        pltpu.SemaphoreType.DMA,
        pltpu.SemaphoreType.DMA,
    ],
)
def _deg_kernel(dst_hbm, ew_hbm, z_hbm, out_hbm, dst_v, ew_v, deg_sh,
                isem, ssem):
    cid = lax.axis_index("c")
    sid = lax.axis_index("s")
    wid = cid * 16 + sid
    pltpu.async_copy(dst_hbm.at[wid], dst_v, isem)
    pltpu.async_copy(ew_hbm.at[wid], ew_v, isem)
    row0 = sid * ROWS_PT
    pltpu.sync_copy(z_hbm.at[pl.ds(row0, ROWS_PT)],
                    deg_sh.at[pl.ds(row0, ROWS_PT)])
    pltpu.make_async_copy(dst_hbm.at[wid], dst_v, isem).wait()
    pltpu.make_async_copy(ew_hbm.at[wid], ew_v, isem).wait()
    plsc.subcore_barrier()

    def fire(c, _):
        pltpu.async_copy(ew_v.at[c], deg_sh.at[dst_v.at[c]], ssem, add=True)
        return 0

    lax.fori_loop(0, NCH, fire, 0)

    def drain(c, _):
        pltpu.make_async_copy(ew_v.at[0], deg_sh.at[dst_v.at[0]], ssem).wait()
        return 0

    lax.fori_loop(0, NCH, drain, 0)
    plsc.subcore_barrier()
    pltpu.sync_copy(deg_sh.at[pl.ds(row0, ROWS_PT)],
                    out_hbm.at[pl.ds(cid * NP + row0, ROWS_PT)])


@functools.partial(
    pl.kernel,
    out_type=jax.ShapeDtypeStruct((2 * NP, F), jnp.float32),
    mesh=_mesh,
    compiler_params=_sc_params,
    scratch_types=[
        pltpu.VMEM((NCH, CH), jnp.int32),
        pltpu.VMEM((NCH, CH), jnp.int32),
        pltpu.VMEM((NCH, CH), jnp.float32),
        pltpu.VMEM((CH, F), jnp.float32),
        pltpu.VMEM((CH, F), jnp.float32),
        pltpu.VMEM((CH, F), jnp.float32),
        pltpu.VMEM((CH, F), jnp.float32),
        pltpu.VMEM_SHARED((NP, F), jnp.float32),
        pltpu.SemaphoreType.DMA,
        [pltpu.SemaphoreType.DMA] * 4,
        [pltpu.SemaphoreType.DMA] * 4,
    ],
)
def _agg_kernel(y_hbm, src_hbm, dst_hbm, ew_hbm, z_hbm, out_hbm,
                src_v, dst_v, ew_v, rows0_v, rows1_v, rows2_v, rows3_v,
                acc_sh, isem, gsem, ssem):
    cid = lax.axis_index("c")
    sid = lax.axis_index("s")
    wid = cid * 16 + sid
    rows = (rows0_v, rows1_v, rows2_v, rows3_v)
    pltpu.async_copy(src_hbm.at[wid], src_v, isem)
    pltpu.async_copy(dst_hbm.at[wid], dst_v, isem)
    pltpu.async_copy(ew_hbm.at[wid], ew_v, isem)
    row0 = sid * ROWS_PT
    pltpu.sync_copy(z_hbm.at[pl.ds(row0, ROWS_PT)],
                    acc_sh.at[pl.ds(row0, ROWS_PT)])
    pltpu.make_async_copy(src_hbm.at[wid], src_v, isem).wait()
    pltpu.make_async_copy(dst_hbm.at[wid], dst_v, isem).wait()
    pltpu.make_async_copy(ew_hbm.at[wid], ew_v, isem).wait()
    plsc.subcore_barrier()

    def gather(c, k):
        pltpu.async_copy(y_hbm.at[src_v.at[c]], rows[k], gsem[k])

    def gather_wait(c, k):
        pltpu.make_async_copy(y_hbm.at[src_v.at[c]], rows[k], gsem[k]).wait()

    def scatter(c, k):
        pltpu.async_copy(rows[k], acc_sh.at[dst_v.at[c]], ssem[k], add=True)

    def scatter_wait(c, k):
        pltpu.make_async_copy(rows[k], acc_sh.at[dst_v.at[c]],
                              ssem[k]).wait()

    def scale(c, k):
        rref = rows[k]

        def scale_grp(g, _):
            wv = ew_v[c, pl.ds(g * 16, 16)]
            base = g * 16
            for l in range(16):
                r = base + l
                rref[r, :] = rref[r, :] * wv[l]
            return 0

        lax.fori_loop(0, CH // 16, scale_grp, 0)

    def step(c, k, wait_ss, prefetch):
        gather_wait(c, k)
        scale(c, k)
        scatter(c, k)
        if prefetch:
            k2 = (k + 2) % 4
            if wait_ss:
                scatter_wait(c, k2)
            gather(c + 2, k2)

    gather(0, 0)
    gather(1, 1)
    step(0, 0, False, True)
    step(1, 1, False, True)
    step(2, 2, True, True)
    step(3, 3, True, True)

    def quad(q, _):
        for k in range(4):
            step(4 * q + k, k, True, True)
        return 0

    lax.fori_loop(1, NCH // 4 - 1, quad, 0)

    step(NCH - 4, 0, True, True)
    step(NCH - 3, 1, True, True)
    step(NCH - 2, 2, False, False)
    step(NCH - 1, 3, False, False)
    for k in range(4):
        scatter_wait(NCH - 4 + k, k)
    plsc.subcore_barrier()
    pltpu.sync_copy(acc_sh.at[pl.ds(row0, ROWS_PT)],
                    out_hbm.at[pl.ds(cid * NP + row0, ROWS_PT)])


def _tc1(x, W1, degp):
    def body(x_ref, w_ref, degp_ref, y1_ref, dinv_ref):
        deg = degp_ref[0:N, :] + degp_ref[NP:NP + N, :] + 1.0
        dinv = lax.rsqrt(deg)
        xw = jnp.dot(x_ref[...], w_ref[...], preferred_element_type=jnp.float32,
                     precision=lax.Precision.HIGHEST)
        y1_ref[...] = xw * dinv
        dinv_ref[...] = dinv

    return pl.pallas_call(
        body,
        out_shape=(jax.ShapeDtypeStruct((N, F), jnp.float32),
                   jax.ShapeDtypeStruct((N, 1), jnp.float32)),
    )(x, W1, degp.reshape(2 * NP, 1))


def _tc2(aggp1, y1, dinv, b1, W2):
    def body(aggp_ref, y1_ref, dinv_ref, b1_ref, w2_ref, y2p_ref):
        agg = aggp_ref[0:N, :] + aggp_ref[NP:NP + N, :] + y1_ref[...]
        h1 = jnp.maximum(agg * dinv_ref[...] + b1_ref[...], 0.0)
        y2 = jnp.dot(h1, w2_ref[...], preferred_element_type=jnp.float32,
                     precision=lax.Precision.HIGHEST)
        y2 = y2 * dinv_ref[...]
        y2p_ref[...] = jnp.concatenate(
            [y2, jnp.zeros((N, F - 4), jnp.float32)], axis=1)

    return pl.pallas_call(
        body,
        out_shape=jax.ShapeDtypeStruct((N, F), jnp.float32),
    )(aggp1, y1, dinv, b1, W2)


def _tc3(aggp2, y2p, dinv, b2, Wl, bl):
    def body(aggp_ref, y2p_ref, dinv_ref, b2_ref, wl_ref, bl_ref, out_ref):
        agg = (aggp_ref[0:N, 0:4] + aggp_ref[NP:NP + N, 0:4]
               + y2p_ref[0:N, 0:4])
        h2 = jnp.maximum(agg * dinv_ref[...] + b2_ref[...], 0.0)
        out = jnp.dot(h2, wl_ref[...], preferred_element_type=jnp.float32,
                      precision=lax.Precision.HIGHEST)
        out_ref[...] = out + bl_ref[...]

    return pl.pallas_call(
        body,
        out_shape=jax.ShapeDtypeStruct((N, 1), jnp.float32),
    )(aggp2, y2p, dinv, b2, Wl, bl)


def _pad_edges(a):
    pad = jnp.zeros((NTILES * EPT - E,), a.dtype)
    return jnp.concatenate([a, pad]).reshape(NTILES, NCH, CH)


def kernel(x, edge_index, edge_attr, W1, b1, W2, b2, Wl, bl):
    src = _pad_edges(edge_index[0])
    dst = _pad_edges(edge_index[1])
    ew3d = _pad_edges(edge_attr)
    z1 = jnp.zeros((NP,), jnp.float32)
    zf = jnp.zeros((NP, F), jnp.float32)
    degp = _deg_kernel(dst, ew3d, z1)
    y1, dinv = _tc1(x, W1, degp)
    aggp1 = _agg_kernel(y1, src, dst, ew3d, zf)
    y2p = _tc2(aggp1, y1, dinv, b1.reshape(1, F), W2)
    aggp2 = _agg_kernel(y2p, src, dst, ew3d, zf)
    return _tc3(aggp2, y2p, dinv, b2.reshape(1, 4), Wl, bl.reshape(1, 1))


# speedup vs baseline: 55.4076x; 1.3875x over previous
import functools

import jax
import jax.numpy as jnp
from jax import lax
from jax.experimental import pallas as pl
from jax.experimental.pallas import tpu as pltpu
from jax.experimental.pallas import tpu_sc as plsc

N = 10000
NP = 10240
E = 320000
NTILES = 32
CH = 128
NCH = 80
EPT = NCH * CH
ROWS_PT = NP // 16
F = 16

_mesh = plsc.VectorSubcoreMesh(core_axis_name="c", subcore_axis_name="s")
_sc_params = pltpu.CompilerParams(use_tc_tiling_on_sc=False)


@functools.partial(
    pl.kernel,
    out_type=jax.ShapeDtypeStruct((2 * NP,), jnp.float32),
    mesh=_mesh,
    compiler_params=_sc_params,
    scratch_types=[
        pltpu.VMEM((NCH, CH), jnp.int32),
        pltpu.VMEM((NCH, CH), jnp.float32),
        pltpu.VMEM_SHARED((NP,), jnp.float32),
        pltpu.SemaphoreType.DMA,
        pltpu.SemaphoreType.DMA,
    ],
)
def _deg_kernel(dst_hbm, ew_hbm, z_hbm, out_hbm, dst_v, ew_v, deg_sh,
                isem, ssem):
    cid = lax.axis_index("c")
    sid = lax.axis_index("s")
    wid = cid * 16 + sid
    pltpu.async_copy(dst_hbm.at[wid], dst_v, isem)
    pltpu.async_copy(ew_hbm.at[wid], ew_v, isem)
    row0 = sid * ROWS_PT
    pltpu.sync_copy(z_hbm.at[pl.ds(row0, ROWS_PT)],
                    deg_sh.at[pl.ds(row0, ROWS_PT)])
    pltpu.make_async_copy(dst_hbm.at[wid], dst_v, isem).wait()
    pltpu.make_async_copy(ew_hbm.at[wid], ew_v, isem).wait()
    plsc.subcore_barrier()

    def fire(c, _):
        pltpu.async_copy(ew_v.at[c], deg_sh.at[dst_v.at[c]], ssem, add=True)
        return 0

    lax.fori_loop(0, NCH, fire, 0)

    def drain(c, _):
        pltpu.make_async_copy(ew_v.at[0], deg_sh.at[dst_v.at[0]], ssem).wait()
        return 0

    lax.fori_loop(0, NCH, drain, 0)
    plsc.subcore_barrier()
    pltpu.sync_copy(deg_sh.at[pl.ds(row0, ROWS_PT)],
                    out_hbm.at[pl.ds(cid * NP + row0, ROWS_PT)])


@functools.partial(
    pl.kernel,
    out_type=jax.ShapeDtypeStruct((2 * NP, F), jnp.float32),
    mesh=_mesh,
    compiler_params=_sc_params,
    scratch_types=[
        pltpu.VMEM((NCH, CH), jnp.int32),
        pltpu.VMEM((NCH, CH), jnp.int32),
        pltpu.VMEM((NCH, CH), jnp.float32),
        pltpu.VMEM((CH, F), jnp.float32),
        pltpu.VMEM((CH, F), jnp.float32),
        pltpu.VMEM((CH, F), jnp.float32),
        pltpu.VMEM((CH, F), jnp.float32),
        pltpu.VMEM_SHARED((NP, F), jnp.float32),
        pltpu.VMEM_SHARED((N, F), jnp.float32),
        pltpu.SemaphoreType.DMA,
        [pltpu.SemaphoreType.DMA] * 4,
        [pltpu.SemaphoreType.DMA] * 4,
    ],
)
def _agg_kernel(y_hbm, src_hbm, dst_hbm, ew_hbm, z_hbm, out_hbm,
                src_v, dst_v, ew_v, rows0_v, rows1_v, rows2_v, rows3_v,
                acc_sh, y_sh, isem, gsem, ssem):
    cid = lax.axis_index("c")
    sid = lax.axis_index("s")
    wid = cid * 16 + sid
    rows = (rows0_v, rows1_v, rows2_v, rows3_v)
    pltpu.async_copy(src_hbm.at[wid], src_v, isem)
    pltpu.async_copy(dst_hbm.at[wid], dst_v, isem)
    pltpu.async_copy(ew_hbm.at[wid], ew_v, isem)
    yr = N // 16
    pltpu.sync_copy(y_hbm.at[pl.ds(sid * yr, yr)],
                    y_sh.at[pl.ds(sid * yr, yr)])
    row0 = sid * ROWS_PT
    pltpu.sync_copy(z_hbm.at[pl.ds(row0, ROWS_PT)],
                    acc_sh.at[pl.ds(row0, ROWS_PT)])
    pltpu.make_async_copy(src_hbm.at[wid], src_v, isem).wait()
    pltpu.make_async_copy(dst_hbm.at[wid], dst_v, isem).wait()
    pltpu.make_async_copy(ew_hbm.at[wid], ew_v, isem).wait()
    plsc.subcore_barrier()

    def gather(c, k):
        pltpu.async_copy(y_sh.at[src_v.at[c]], rows[k], gsem[k])

    def gather_wait(c, k):
        pltpu.make_async_copy(y_sh.at[src_v.at[c]], rows[k], gsem[k]).wait()

    def scatter(c, k):
        pltpu.async_copy(rows[k], acc_sh.at[dst_v.at[c]], ssem[k], add=True)

    def scatter_wait(c, k):
        pltpu.make_async_copy(rows[k], acc_sh.at[dst_v.at[c]],
                              ssem[k]).wait()

    def scale(c, k):
        rref = rows[k]

        def scale_grp(g, _):
            wv = ew_v[c, pl.ds(g * 16, 16)]
            base = g * 16
            for l in range(16):
                r = base + l
                rref[r, :] = rref[r, :] * wv[l]
            return 0

        lax.fori_loop(0, CH // 16, scale_grp, 0)

    def step(c, k, wait_ss, prefetch):
        gather_wait(c, k)
        scale(c, k)
        scatter(c, k)
        if prefetch:
            k2 = (k + 2) % 4
            if wait_ss:
                scatter_wait(c, k2)
            gather(c + 2, k2)

    gather(0, 0)
    gather(1, 1)
    step(0, 0, False, True)
    step(1, 1, False, True)
    step(2, 2, True, True)
    step(3, 3, True, True)

    def quad(q, _):
        for k in range(4):
            step(4 * q + k, k, True, True)
        return 0

    lax.fori_loop(1, NCH // 4 - 1, quad, 0)

    step(NCH - 4, 0, True, True)
    step(NCH - 3, 1, True, True)
    step(NCH - 2, 2, False, False)
    step(NCH - 1, 3, False, False)
    for k in range(4):
        scatter_wait(NCH - 4 + k, k)
    plsc.subcore_barrier()
    pltpu.sync_copy(acc_sh.at[pl.ds(row0, ROWS_PT)],
                    out_hbm.at[pl.ds(cid * NP + row0, ROWS_PT)])


def _tc1(x, W1, degp):
    def body(x_ref, w_ref, degp_ref, y1_ref, dinv_ref):
        deg = degp_ref[0:N, :] + degp_ref[NP:NP + N, :] + 1.0
        dinv = lax.rsqrt(deg)
        xw = jnp.dot(x_ref[...], w_ref[...], preferred_element_type=jnp.float32,
                     precision=lax.Precision.HIGHEST)
        y1_ref[...] = xw * dinv
        dinv_ref[...] = dinv

    return pl.pallas_call(
        body,
        out_shape=(jax.ShapeDtypeStruct((N, F), jnp.float32),
                   jax.ShapeDtypeStruct((N, 1), jnp.float32)),
    )(x, W1, degp.reshape(2 * NP, 1))


def _tc2(aggp1, y1, dinv, b1, W2):
    def body(aggp_ref, y1_ref, dinv_ref, b1_ref, w2_ref, y2p_ref):
        agg = aggp_ref[0:N, :] + aggp_ref[NP:NP + N, :] + y1_ref[...]
        h1 = jnp.maximum(agg * dinv_ref[...] + b1_ref[...], 0.0)
        y2 = jnp.dot(h1, w2_ref[...], preferred_element_type=jnp.float32,
                     precision=lax.Precision.HIGHEST)
        y2 = y2 * dinv_ref[...]
        y2p_ref[...] = jnp.concatenate(
            [y2, jnp.zeros((N, F - 4), jnp.float32)], axis=1)

    return pl.pallas_call(
        body,
        out_shape=jax.ShapeDtypeStruct((N, F), jnp.float32),
    )(aggp1, y1, dinv, b1, W2)


def _tc3(aggp2, y2p, dinv, b2, Wl, bl):
    def body(aggp_ref, y2p_ref, dinv_ref, b2_ref, wl_ref, bl_ref, out_ref):
        agg = (aggp_ref[0:N, 0:4] + aggp_ref[NP:NP + N, 0:4]
               + y2p_ref[0:N, 0:4])
        h2 = jnp.maximum(agg * dinv_ref[...] + b2_ref[...], 0.0)
        out = jnp.dot(h2, wl_ref[...], preferred_element_type=jnp.float32,
                      precision=lax.Precision.HIGHEST)
        out_ref[...] = out + bl_ref[...]

    return pl.pallas_call(
        body,
        out_shape=jax.ShapeDtypeStruct((N, 1), jnp.float32),
    )(aggp2, y2p, dinv, b2, Wl, bl)


def _pad_edges(a):
    pad = jnp.zeros((NTILES * EPT - E,), a.dtype)
    return jnp.concatenate([a, pad]).reshape(NTILES, NCH, CH)


def kernel(x, edge_index, edge_attr, W1, b1, W2, b2, Wl, bl):
    src = _pad_edges(edge_index[0])
    dst = _pad_edges(edge_index[1])
    ew3d = _pad_edges(edge_attr)
    z1 = jnp.zeros((NP,), jnp.float32)
    zf = jnp.zeros((NP, F), jnp.float32)
    degp = _deg_kernel(dst, ew3d, z1)
    y1, dinv = _tc1(x, W1, degp)
    aggp1 = _agg_kernel(y1, src, dst, ew3d, zf)
    y2p = _tc2(aggp1, y1, dinv, b1.reshape(1, F), W2)
    aggp2 = _agg_kernel(y2p, src, dst, ew3d, zf)
    return _tc3(aggp2, y2p, dinv, b2.reshape(1, 4), Wl, bl.reshape(1, 1))
